# baseline (speedup 1.0000x reference)
import functools

import jax
import jax.numpy as jnp
from jax import lax
from jax.experimental import pallas as pl
from jax.experimental.pallas import tpu as pltpu
from jax.experimental.pallas import tpu_sc as plsc

_KERNEL_NUM = 64
_SIZE = 1048576
_LANES = 16
_NSUB = 8
_NBUF = 2


def _make_sc_kernel():
    info = plsc.get_sparse_core_info()
    num_workers = info.num_cores * info.num_subcores
    chunk = _SIZE // num_workers
    sub = chunk // _NSUB

    mesh = plsc.VectorSubcoreMesh(core_axis_name="c", subcore_axis_name="s")

    @functools.partial(
        pl.kernel,
        out_type=jax.ShapeDtypeStruct((1, _SIZE), jnp.float32),
        mesh=mesh,
        scratch_types=[
            pltpu.VMEM((_LANES,), jnp.float32),
            pltpu.VMEM((_NBUF, 2, sub), jnp.float32),
            pltpu.VMEM((_NBUF, sub), jnp.float32),
            [pltpu.SemaphoreType.DMA] * _NBUF,
            [pltpu.SemaphoreType.DMA] * _NBUF,
            [pltpu.SemaphoreType.DMA] * _NBUF,
        ],
    )
    def blend(lam_hbm, const_hbm, pivots_hbm, out_hbm, lam_v, ibuf, obuf,
              lsems, rsems, osems):
        del pivots_hbm
        wid = lax.axis_index("s") * info.num_cores + lax.axis_index("c")
        base = wid * chunk

        pltpu.sync_copy(lam_hbm, lam_v.at[pl.ds(0, 1)])
        lam_s = lam_v[...][0] * jnp.float32(0.99999)

        scaled = lam_s * jnp.float32(_KERNEL_NUM - 1)
        left = scaled.astype(jnp.int32)
        left = jnp.minimum(jnp.maximum(left, 0), _KERNEL_NUM - 2)
        dist = (left.astype(jnp.float32) + jnp.float32(1.0)) - scaled
        one_minus = jnp.float32(1.0) - dist

        def start_in(g, slot):
            col = pl.ds(base + g * sub, sub)
            pltpu.async_copy(const_hbm.at[left, col],
                             ibuf.at[slot, 0], lsems[slot])
            pltpu.async_copy(const_hbm.at[left + 1, col],
                             ibuf.at[slot, 1], rsems[slot])

        start_in(0, 0)

        for g in range(_NSUB):
            slot = g % _NBUF
            nxt = (g + 1) % _NBUF
            if g + 1 < _NSUB:
                start_in(g + 1, nxt)
            pltpu.make_async_copy(
                const_hbm.at[left, pl.ds(base, sub)],
                ibuf.at[slot, 0], lsems[slot]).wait()
            pltpu.make_async_copy(
                const_hbm.at[left + 1, pl.ds(base, sub)],
                ibuf.at[slot, 1], rsems[slot]).wait()
            if g >= _NBUF:
                pltpu.make_async_copy(
                    obuf.at[slot],
                    out_hbm.at[0, pl.ds(base, sub)], osems[slot]).wait()

            @plsc.parallel_loop(0, sub, step=_LANES, unroll=8)
            def _(i):
                sl = pl.ds(i, _LANES)
                obuf[slot, sl] = (ibuf[slot, 0, sl] * dist
                                  + ibuf[slot, 1, sl] * one_minus)

            pltpu.async_copy(
                obuf.at[slot],
                out_hbm.at[0, pl.ds(base + g * sub, sub)], osems[slot])

        for slot in range(_NBUF):
            pltpu.make_async_copy(
                obuf.at[slot],
                out_hbm.at[0, pl.ds(base, sub)], osems[slot]).wait()

    return blend


_blend = _make_sc_kernel()


def kernel(lam, const, pivots):
    return _blend(lam, const, pivots)


# speedup vs baseline: 1.0440x; 1.0440x over previous
import functools

import jax
import jax.numpy as jnp
from jax import lax
from jax.experimental import pallas as pl
from jax.experimental.pallas import tpu as pltpu
from jax.experimental.pallas import tpu_sc as plsc

_KERNEL_NUM = 64
_SIZE = 1048576
_LANES = 16
_NSUB = 4
_NBUF = 2


def _make_sc_kernel():
    info = plsc.get_sparse_core_info()
    num_workers = info.num_cores * info.num_subcores
    chunk = _SIZE // num_workers
    sub = chunk // _NSUB

    mesh = plsc.VectorSubcoreMesh(core_axis_name="c", subcore_axis_name="s")

    @functools.partial(
        pl.kernel,
        out_type=jax.ShapeDtypeStruct((1, _SIZE), jnp.float32),
        mesh=mesh,
        scratch_types=[
            pltpu.VMEM((_LANES,), jnp.float32),
            pltpu.VMEM((_NBUF, 2, sub), jnp.float32),
            pltpu.VMEM((_NBUF, sub), jnp.float32),
            [pltpu.SemaphoreType.DMA] * _NBUF,
            [pltpu.SemaphoreType.DMA] * _NBUF,
            [pltpu.SemaphoreType.DMA] * _NBUF,
        ],
    )
    def blend(lam_hbm, const_hbm, pivots_hbm, out_hbm, lam_v, ibuf, obuf,
              lsems, rsems, osems):
        del pivots_hbm
        wid = lax.axis_index("s") * info.num_cores + lax.axis_index("c")
        base = wid * chunk

        pltpu.sync_copy(lam_hbm, lam_v.at[pl.ds(0, 1)])
        lam_s = lam_v[...][0] * jnp.float32(0.99999)

        scaled = lam_s * jnp.float32(_KERNEL_NUM - 1)
        left = scaled.astype(jnp.int32)
        left = jnp.minimum(jnp.maximum(left, 0), _KERNEL_NUM - 2)
        dist = (left.astype(jnp.float32) + jnp.float32(1.0)) - scaled
        one_minus = jnp.float32(1.0) - dist

        def start_in(g, slot):
            col = pl.ds(base + g * sub, sub)
            pltpu.async_copy(const_hbm.at[left, col],
                             ibuf.at[slot, 0], lsems[slot])
            pltpu.async_copy(const_hbm.at[left + 1, col],
                             ibuf.at[slot, 1], rsems[slot])

        start_in(0, 0)

        for g in range(_NSUB):
            slot = g % _NBUF
            nxt = (g + 1) % _NBUF
            if g + 1 < _NSUB:
                start_in(g + 1, nxt)
            pltpu.make_async_copy(
                const_hbm.at[left, pl.ds(base, sub)],
                ibuf.at[slot, 0], lsems[slot]).wait()
            pltpu.make_async_copy(
                const_hbm.at[left + 1, pl.ds(base, sub)],
                ibuf.at[slot, 1], rsems[slot]).wait()
            if g >= _NBUF:
                pltpu.make_async_copy(
                    obuf.at[slot],
                    out_hbm.at[0, pl.ds(base, sub)], osems[slot]).wait()

            @plsc.parallel_loop(0, sub, step=_LANES, unroll=8)
            def _(i):
                sl = pl.ds(i, _LANES)
                obuf[slot, sl] = (ibuf[slot, 0, sl] * dist
                                  + ibuf[slot, 1, sl] * one_minus)

            pltpu.async_copy(
                obuf.at[slot],
                out_hbm.at[0, pl.ds(base + g * sub, sub)], osems[slot])

        for slot in range(_NBUF):
            pltpu.make_async_copy(
                obuf.at[slot],
                out_hbm.at[0, pl.ds(base, sub)], osems[slot]).wait()

    return blend


_blend = _make_sc_kernel()


def kernel(lam, const, pivots):
    return _blend(lam, const, pivots)
